# Initial kernel scaffold; baseline (speedup 1.0000x reference)
import jax
import jax.numpy as jnp
from jax.experimental import pallas as pl


def kernel(states, actions, index, W1, b1, W2, b2, W3, b3):
    raise NotImplementedError("write your pallas kernel here")


# speedup vs baseline: 2.2171x; 2.2171x over previous
import functools

import jax
import jax.numpy as jnp
from jax.experimental import pallas as pl
from jax.experimental.pallas import tpu as pltpu

NUM_MODELS = 8
STATE_DIM = 256
AC_DIM = 64
HIDDEN = 512
BATCH = 4096
IN_DIM = STATE_DIM + AC_DIM


def _dense_kernel(idx_ref, x_ref, w1_ref, b1_ref, w2_ref, b2_ref, w3_ref,
                  b3_ref, out_ref):
    e = pl.program_id(0)
    x = x_ref[...]
    h = jnp.maximum(
        jnp.dot(x, w1_ref[0], preferred_element_type=jnp.float32)
        + b1_ref[0, 0], 0.0).astype(jnp.bfloat16)
    h = jnp.maximum(
        jnp.dot(h, w2_ref[0], preferred_element_type=jnp.float32)
        + b2_ref[0, 0], 0.0).astype(jnp.bfloat16)
    o = jnp.dot(h, w3_ref[0], preferred_element_type=jnp.float32) + b3_ref[0, 0]
    mask = idx_ref[...] == e

    @pl.when(e == 0)
    def _():
        out_ref[...] = jnp.where(mask, o, 0.0)

    @pl.when(e > 0)
    def _():
        out_ref[...] = jnp.where(mask, o, out_ref[...])


@jax.jit
def kernel(states, actions, index, W1, b1, W2, b2, W3, b3):
    x = jnp.concatenate([states, actions], axis=-1).astype(jnp.bfloat16)
    idx = index.astype(jnp.int32).reshape(BATCH, 1)
    grid = (NUM_MODELS,)
    out = pl.pallas_call(
        _dense_kernel,
        grid=grid,
        in_specs=[
            pl.BlockSpec((BATCH, 1), lambda e: (0, 0)),
            pl.BlockSpec((BATCH, IN_DIM), lambda e: (0, 0)),
            pl.BlockSpec((1, IN_DIM, HIDDEN), lambda e: (e, 0, 0)),
            pl.BlockSpec((1, 1, HIDDEN), lambda e: (e, 0, 0)),
            pl.BlockSpec((1, HIDDEN, HIDDEN), lambda e: (e, 0, 0)),
            pl.BlockSpec((1, 1, HIDDEN), lambda e: (e, 0, 0)),
            pl.BlockSpec((1, HIDDEN, STATE_DIM), lambda e: (e, 0, 0)),
            pl.BlockSpec((1, 1, STATE_DIM), lambda e: (e, 0, 0)),
        ],
        out_specs=pl.BlockSpec((BATCH, STATE_DIM), lambda e: (0, 0)),
        out_shape=jax.ShapeDtypeStruct((BATCH, STATE_DIM), jnp.float32),
    )(idx, x, W1.astype(jnp.bfloat16), b1.reshape(NUM_MODELS, 1, HIDDEN),
      W2.astype(jnp.bfloat16), b2.reshape(NUM_MODELS, 1, HIDDEN),
      W3.astype(jnp.bfloat16), b3.reshape(NUM_MODELS, 1, STATE_DIM))
    return out
